# baseline (speedup 1.0000x reference)
import jax
import jax.numpy as jnp
from jax import lax
from jax.experimental import pallas as pl
from jax.experimental.pallas import tpu as pltpu
from jax.experimental.pallas import tpu_sc as plsc

B, T, D = 8, 4096, 2048
K = 10
L = 16
NQ = 4
SLEN = T // NQ
DQ = D // NQ
NEG = float("-inf")
INT_MAX = 2147483647

PAIRS = [(i, j) for i in range(K) for j in range(i, K)]
DIAG_SLOTS = [p for p, (i, j) in enumerate(PAIRS) if i == j]
NP_ = len(PAIRS)
GROUPS = [
    [p for p in PAIRS if p[0] < 2],
    [p for p in PAIRS if 2 <= p[0] < 5],
    [p for p in PAIRS if p[0] >= 5],
]


def _sc_diversity(feat_hbm, scores_hbm, c_val, c_idx, stage2, stage3, out_hbm,
                  scores_v, idx_v, rows_v, cv_v, ci_v, mat_v, t16_v, t4x16_v,
                  comb_v, red_v, sem):
  c = lax.axis_index("c")
  s = lax.axis_index("s")
  lanes = lax.iota(jnp.int32, L)
  bl = s // NQ
  q = s % NQ
  b = NQ * c + bl

  pltpu.sync_copy(scores_hbm.at[b, pl.ds(q * SLEN, SLEN)], scores_v)
  mxv = jnp.full((L,), NEG, jnp.float32)
  fiv = jnp.zeros((L,), jnp.int32)
  for kk in range(K):
    def chunk_body(i, carry):
      mv, iv = carry
      v = scores_v[pl.ds(i * L, L)]
      ids = lanes + i * L
      gt = v > mv
      return jnp.where(gt, v, mv), jnp.where(gt, ids, iv)
    mv, iv = lax.fori_loop(
        0, SLEN // L, chunk_body,
        (jnp.full((L,), NEG, jnp.float32), jnp.zeros((L,), jnp.int32)),
        unroll=8)
    mx = jnp.max(mv)
    cand = jnp.where(mv == mx, iv, INT_MAX)
    idx = jnp.min(cand)
    mxv = jnp.where(lanes == kk, mx, mxv)
    fiv = jnp.where(lanes == kk, idx + q * SLEN, fiv)
    plsc.store_scatter(scores_v, [jnp.full((L,), idx, jnp.int32)],
                       jnp.full((L,), NEG, jnp.float32), mask=lanes == 0)
  t16_v[...] = mxv
  pltpu.sync_copy(t16_v, c_val.at[c, s])
  idx_v[...] = fiv
  pltpu.sync_copy(idx_v, c_idx.at[c, s])

  plsc.subcore_barrier()

  pltpu.sync_copy(c_val.at[c, pl.ds(NQ * bl, NQ)], cv_v)
  pltpu.sync_copy(c_idx.at[c, pl.ds(NQ * bl, NQ)], ci_v)
  vs = [cv_v[r, :] for r in range(NQ)]
  ix = [ci_v[r, :] for r in range(NQ)]
  fiv = jnp.zeros((L,), jnp.int32)
  for kk in range(K):
    def comb(v1, i1, v2, i2):
      take1 = (v1 > v2) | ((v1 == v2) & (i1 < i2))
      return jnp.where(take1, v1, v2), jnp.where(take1, i1, i2)
    va, ia = comb(vs[0], ix[0], vs[1], ix[1])
    vb, ib = comb(vs[2], ix[2], vs[3], ix[3])
    vm, im = comb(va, ia, vb, ib)
    mx = jnp.max(vm)
    cand = jnp.where(vm == mx, im, INT_MAX)
    idx = jnp.min(cand)
    fiv = jnp.where(lanes == kk, idx, fiv)
    idx_b = jnp.full((L,), idx, jnp.int32)
    for r in range(NQ):
      vs[r] = jnp.where(ix[r] == idx_b, NEG, vs[r])

  idx_v[...] = (jnp.where(lanes < K, fiv, 0) + b * T) * NQ + q
  pltpu.async_copy(feat_hbm.at[idx_v], rows_v, sem).wait()

  accs = []
  for grp in GROUPS:
    rows_needed = sorted({r for p in grp for r in p})
    def grp_body(ci, carry, grp=grp, rows_needed=rows_needed):
      base = ci * L
      v = {r: rows_v[r, pl.ds(base, L)] for r in rows_needed}
      return tuple(a + v[i] * v[j] for a, (i, j) in zip(carry, grp))
    init = tuple(jnp.zeros((L,), jnp.float32) for _ in grp)
    accs.extend(lax.fori_loop(0, DQ // L, grp_body, init, unroll=2))
  for p in range(NP_):
    mat_v[p, :] = accs[p]
  zero = jnp.zeros((L,), jnp.float32)
  for p in range(NP_, 64):
    mat_v[p, :] = zero
  for blk in range(4):
    row_ids = lanes + blk * L
    dots = jnp.zeros((L,), jnp.float32)
    for l in range(L):
      dots = dots + plsc.load_gather(
          mat_v, [row_ids, jnp.full((L,), l, jnp.int32)])
    t4x16_v[blk, :] = dots
  pltpu.sync_copy(t4x16_v, stage2.at[c, s])

  plsc.subcore_barrier()

  @pl.when(q == 0)
  def _batch_combine():
    pltpu.sync_copy(stage2.at[c, pl.ds(NQ * bl, NQ)], comb_v)
    inv = 1.0 / (B * K * K)
    tot = jnp.zeros((L,), jnp.float32)
    for blk in range(4):
      d = (comb_v[0, blk, :] + comb_v[1, blk, :] +
           comb_v[2, blk, :] + comb_v[3, blk, :])
      diag_here = [p - blk * L for p in DIAG_SLOTS if blk * L <= p < (blk + 1) * L]
      dm = lanes < 0
      for dpos in diag_here:
        dm = dm | (lanes == dpos)
      off = jnp.where(dm, 1.0, 0.0)
      w = jnp.where(dm, 1.0, 2.0)
      tot = tot + w * jnp.abs(d - off)
    part = jnp.sum(tot) * inv
    t16_v[...] = jnp.full((L,), part)
    pltpu.sync_copy(t16_v, stage3.at[c, bl])

  plsc.subcore_barrier()

  @pl.when(s == 0)
  def _core_reduce():
    pltpu.sync_copy(stage3.at[c], red_v)
    tot = red_v[0, :] + red_v[1, :] + red_v[2, :] + red_v[3, :]
    t16_v[...] = tot
    pltpu.sync_copy(t16_v, out_hbm.at[c])


@jax.jit
def kernel(features, scores):
  table = features.reshape(B * T * NQ, DQ)
  mesh = plsc.VectorSubcoreMesh(core_axis_name="c", subcore_axis_name="s",
                                num_cores=2, num_subcores=16)
  outs = pl.kernel(
      _sc_diversity,
      out_type=(
          jax.ShapeDtypeStruct((2, 16, L), jnp.float32),
          jax.ShapeDtypeStruct((2, 16, L), jnp.int32),
          jax.ShapeDtypeStruct((2, 16, 4, L), jnp.float32),
          jax.ShapeDtypeStruct((2, 4, L), jnp.float32),
          jax.ShapeDtypeStruct((2, L), jnp.float32),
      ),
      mesh=mesh,
      compiler_params=pltpu.CompilerParams(needs_layout_passes=False),
      scratch_types=[
          pltpu.VMEM((SLEN,), jnp.float32),
          pltpu.VMEM((L,), jnp.int32),
          pltpu.VMEM((L, DQ), jnp.float32),
          pltpu.VMEM((NQ, L), jnp.float32),
          pltpu.VMEM((NQ, L), jnp.int32),
          pltpu.VMEM((64, L), jnp.float32),
          pltpu.VMEM((L,), jnp.float32),
          pltpu.VMEM((4, L), jnp.float32),
          pltpu.VMEM((NQ, 4, L), jnp.float32),
          pltpu.VMEM((4, L), jnp.float32),
          pltpu.SemaphoreType.DMA,
      ],
  )(table, scores)
  out = outs[4]
  return out[0, 0] + out[1, 0]


# speedup vs baseline: 9.3776x; 9.3776x over previous
import jax
import jax.numpy as jnp
from jax import lax
from jax.experimental import pallas as pl
from jax.experimental.pallas import tpu as pltpu
from jax.experimental.pallas import tpu_sc as plsc

B, T, D = 8, 4096, 2048
K = 10
L = 16
NQ = 4
SLEN = T // NQ
DQ = D // NQ
NEG = float("-inf")
INT_MAX = 2147483647

PAIRS = [(i, j) for i in range(K) for j in range(i, K)]
DIAG_SLOTS = [p for p, (i, j) in enumerate(PAIRS) if i == j]
NP_ = len(PAIRS)
GROUPS = [
    [p for p in PAIRS if p[0] < 2],
    [p for p in PAIRS if 2 <= p[0] < 5],
    [p for p in PAIRS if p[0] >= 5],
]


def _sc_diversity(feat_hbm, scores_hbm, c_val, c_idx, stage2, stage3, out_hbm,
                  scores_v, idx_v, rows_v, cv_v, ci_v, mat_v, t16_v, t4x16_v,
                  comb_v, red_v, sem):
  c = lax.axis_index("c")
  s = lax.axis_index("s")
  lanes = lax.iota(jnp.int32, L)
  bl = s // NQ
  q = s % NQ
  b = NQ * c + bl

  pltpu.sync_copy(scores_hbm.at[b, pl.ds(q * SLEN, SLEN)], scores_v)
  mxv = jnp.full((L,), NEG, jnp.float32)
  fiv = jnp.zeros((L,), jnp.int32)
  for kk in range(K):
    def chunk_body(i, carry):
      mv, iv = carry
      v = scores_v[pl.ds(i * L, L)]
      ids = lanes + i * L
      gt = v > mv
      return jnp.where(gt, v, mv), jnp.where(gt, ids, iv)
    mv, iv = lax.fori_loop(
        0, SLEN // L, chunk_body,
        (jnp.full((L,), NEG, jnp.float32), jnp.zeros((L,), jnp.int32)),
        unroll=8)
    mx = jnp.max(mv)
    cand = jnp.where(mv == mx, iv, INT_MAX)
    idx = jnp.min(cand)
    mxv = jnp.where(lanes == kk, mx, mxv)
    fiv = jnp.where(lanes == kk, idx + q * SLEN, fiv)
    plsc.store_scatter(scores_v, [jnp.full((L,), idx, jnp.int32)],
                       jnp.full((L,), NEG, jnp.float32), mask=lanes == 0)
  t16_v[...] = mxv
  pltpu.sync_copy(t16_v, c_val.at[c, s])
  idx_v[...] = fiv
  pltpu.sync_copy(idx_v, c_idx.at[c, s])

  plsc.subcore_barrier()

  pltpu.sync_copy(c_val.at[c, pl.ds(NQ * bl, NQ)], cv_v)
  pltpu.sync_copy(c_idx.at[c, pl.ds(NQ * bl, NQ)], ci_v)
  vs = [cv_v[r, :] for r in range(NQ)]
  ix = [ci_v[r, :] for r in range(NQ)]
  fiv = jnp.zeros((L,), jnp.int32)
  for kk in range(K):
    def comb(v1, i1, v2, i2):
      take1 = (v1 > v2) | ((v1 == v2) & (i1 < i2))
      return jnp.where(take1, v1, v2), jnp.where(take1, i1, i2)
    va, ia = comb(vs[0], ix[0], vs[1], ix[1])
    vb, ib = comb(vs[2], ix[2], vs[3], ix[3])
    vm, im = comb(va, ia, vb, ib)
    mx = jnp.max(vm)
    cand = jnp.where(vm == mx, im, INT_MAX)
    idx = jnp.min(cand)
    fiv = jnp.where(lanes == kk, idx, fiv)
    idx_b = jnp.full((L,), idx, jnp.int32)
    for r in range(NQ):
      vs[r] = jnp.where(ix[r] == idx_b, NEG, vs[r])

  idx_v[...] = jnp.where(lanes < K, fiv, 0) + b * T
  pltpu.async_copy(feat_hbm.at[idx_v, pl.ds(q * DQ, DQ)], rows_v, sem).wait()

  accs = []
  for grp in GROUPS:
    rows_needed = sorted({r for p in grp for r in p})
    def grp_body(ci, carry, grp=grp, rows_needed=rows_needed):
      base = ci * L
      v = {r: rows_v[r, pl.ds(base, L)] for r in rows_needed}
      return tuple(a + v[i] * v[j] for a, (i, j) in zip(carry, grp))
    init = tuple(jnp.zeros((L,), jnp.float32) for _ in grp)
    accs.extend(lax.fori_loop(0, DQ // L, grp_body, init, unroll=2))
  for p in range(NP_):
    mat_v[p, :] = accs[p]
  zero = jnp.zeros((L,), jnp.float32)
  for p in range(NP_, 64):
    mat_v[p, :] = zero
  for blk in range(4):
    row_ids = lanes + blk * L
    dots = jnp.zeros((L,), jnp.float32)
    for l in range(L):
      dots = dots + plsc.load_gather(
          mat_v, [row_ids, jnp.full((L,), l, jnp.int32)])
    t4x16_v[blk, :] = dots
  pltpu.sync_copy(t4x16_v, stage2.at[c, s])

  plsc.subcore_barrier()

  @pl.when(q == 0)
  def _batch_combine():
    pltpu.sync_copy(stage2.at[c, pl.ds(NQ * bl, NQ)], comb_v)
    inv = 1.0 / (B * K * K)
    tot = jnp.zeros((L,), jnp.float32)
    for blk in range(4):
      d = (comb_v[0, blk, :] + comb_v[1, blk, :] +
           comb_v[2, blk, :] + comb_v[3, blk, :])
      diag_here = [p - blk * L for p in DIAG_SLOTS if blk * L <= p < (blk + 1) * L]
      dm = lanes < 0
      for dpos in diag_here:
        dm = dm | (lanes == dpos)
      off = jnp.where(dm, 1.0, 0.0)
      w = jnp.where(dm, 1.0, 2.0)
      tot = tot + w * jnp.abs(d - off)
    part = jnp.sum(tot) * inv
    t16_v[...] = jnp.full((L,), part)
    pltpu.sync_copy(t16_v, stage3.at[c, bl])

  plsc.subcore_barrier()

  @pl.when(s == 0)
  def _core_reduce():
    pltpu.sync_copy(stage3.at[c], red_v)
    tot = red_v[0, :] + red_v[1, :] + red_v[2, :] + red_v[3, :]
    t16_v[...] = tot
    pltpu.sync_copy(t16_v, out_hbm.at[c])


@jax.jit
def kernel(features, scores):
  table = features.reshape(B * T, D)
  mesh = plsc.VectorSubcoreMesh(core_axis_name="c", subcore_axis_name="s",
                                num_cores=2, num_subcores=16)
  outs = pl.kernel(
      _sc_diversity,
      out_type=(
          jax.ShapeDtypeStruct((2, 16, L), jnp.float32),
          jax.ShapeDtypeStruct((2, 16, L), jnp.int32),
          jax.ShapeDtypeStruct((2, 16, 4, L), jnp.float32),
          jax.ShapeDtypeStruct((2, 4, L), jnp.float32),
          jax.ShapeDtypeStruct((2, L), jnp.float32),
      ),
      mesh=mesh,
      compiler_params=pltpu.CompilerParams(needs_layout_passes=False),
      scratch_types=[
          pltpu.VMEM((SLEN,), jnp.float32),
          pltpu.VMEM((L,), jnp.int32),
          pltpu.VMEM((L, DQ), jnp.float32),
          pltpu.VMEM((NQ, L), jnp.float32),
          pltpu.VMEM((NQ, L), jnp.int32),
          pltpu.VMEM((64, L), jnp.float32),
          pltpu.VMEM((L,), jnp.float32),
          pltpu.VMEM((4, L), jnp.float32),
          pltpu.VMEM((NQ, 4, L), jnp.float32),
          pltpu.VMEM((4, L), jnp.float32),
          pltpu.SemaphoreType.DMA,
      ],
  )(table, scores)
  out = outs[4]
  return out[0, 0] + out[1, 0]


# speedup vs baseline: 9.6086x; 1.0246x over previous
import jax
import jax.numpy as jnp
from jax import lax
from jax.experimental import pallas as pl
from jax.experimental.pallas import tpu as pltpu
from jax.experimental.pallas import tpu_sc as plsc

B, T, D = 8, 4096, 2048
K = 10
L = 16
NQ = 4
SLEN = T // NQ
DQ = D // NQ
NEG = float("-inf")
INT_MAX = 2147483647

PAIRS = [(i, j) for i in range(K) for j in range(i, K)]
DIAG_SLOTS = [p for p, (i, j) in enumerate(PAIRS) if i == j]
NP_ = len(PAIRS)
GROUPS = [
    [p for p in PAIRS if p[0] < 2],
    [p for p in PAIRS if 2 <= p[0] < 5],
    [p for p in PAIRS if p[0] >= 5],
]


def _sc_diversity(feat_hbm, scores_hbm, c_val, c_idx, stage2, out_hbm,
                  scores_v, idx_v, rows_v, cv_v, ci_v, mat_v, t16_v, t4x16_v,
                  comb_v, sem):
  c = lax.axis_index("c")
  s = lax.axis_index("s")
  lanes = lax.iota(jnp.int32, L)
  bl = s // NQ
  q = s % NQ
  b = NQ * c + bl

  pltpu.sync_copy(scores_hbm.at[b, pl.ds(q * SLEN, SLEN)], scores_v)
  mxv = jnp.full((L,), NEG, jnp.float32)
  fiv = jnp.zeros((L,), jnp.int32)
  for kk in range(K):
    def chunk_body(i, carry):
      mv, iv = carry
      v = scores_v[pl.ds(i * L, L)]
      ids = lanes + i * L
      gt = v > mv
      return jnp.where(gt, v, mv), jnp.where(gt, ids, iv)
    mv, iv = lax.fori_loop(
        0, SLEN // L, chunk_body,
        (jnp.full((L,), NEG, jnp.float32), jnp.zeros((L,), jnp.int32)),
        unroll=4)
    mx = jnp.max(mv)
    cand = jnp.where(mv == mx, iv, INT_MAX)
    idx = jnp.min(cand)
    mxv = jnp.where(lanes == kk, mx, mxv)
    fiv = jnp.where(lanes == kk, idx + q * SLEN, fiv)
    plsc.store_scatter(scores_v, [jnp.full((L,), idx, jnp.int32)],
                       jnp.full((L,), NEG, jnp.float32), mask=lanes == 0)
  t16_v[...] = mxv
  pltpu.sync_copy(t16_v, c_val.at[c, s])
  idx_v[...] = fiv
  pltpu.sync_copy(idx_v, c_idx.at[c, s])

  plsc.subcore_barrier()

  pltpu.sync_copy(c_val.at[c, pl.ds(NQ * bl, NQ)], cv_v)
  pltpu.sync_copy(c_idx.at[c, pl.ds(NQ * bl, NQ)], ci_v)
  vs = [cv_v[r, :] for r in range(NQ)]
  ix = [ci_v[r, :] for r in range(NQ)]
  fiv = jnp.zeros((L,), jnp.int32)
  for kk in range(K):
    def comb(v1, i1, v2, i2):
      take1 = (v1 > v2) | ((v1 == v2) & (i1 < i2))
      return jnp.where(take1, v1, v2), jnp.where(take1, i1, i2)
    va, ia = comb(vs[0], ix[0], vs[1], ix[1])
    vb, ib = comb(vs[2], ix[2], vs[3], ix[3])
    vm, im = comb(va, ia, vb, ib)
    mx = jnp.max(vm)
    cand = jnp.where(vm == mx, im, INT_MAX)
    idx = jnp.min(cand)
    fiv = jnp.where(lanes == kk, idx, fiv)
    idx_b = jnp.full((L,), idx, jnp.int32)
    for r in range(NQ):
      vs[r] = jnp.where(ix[r] == idx_b, NEG, vs[r])

  idx_v[...] = jnp.where(lanes < K, fiv, 0) + b * T
  pltpu.async_copy(feat_hbm.at[idx_v, pl.ds(q * DQ, DQ)], rows_v, sem).wait()

  accs = []
  for grp in GROUPS:
    rows_needed = sorted({r for p in grp for r in p})
    def grp_body(ci, carry, grp=grp, rows_needed=rows_needed):
      base = ci * L
      v = {r: rows_v[r, pl.ds(base, L)] for r in rows_needed}
      return tuple(a + v[i] * v[j] for a, (i, j) in zip(carry, grp))
    init = tuple(jnp.zeros((L,), jnp.float32) for _ in grp)
    accs.extend(lax.fori_loop(0, DQ // L, grp_body, init, unroll=2))
  for p in range(NP_):
    mat_v[p, :] = accs[p]
  zero = jnp.zeros((L,), jnp.float32)
  for p in range(NP_, 64):
    mat_v[p, :] = zero
  for blk in range(4):
    row_ids = lanes + blk * L
    dots = jnp.zeros((L,), jnp.float32)
    for l in range(L):
      dots = dots + plsc.load_gather(
          mat_v, [row_ids, jnp.full((L,), l, jnp.int32)])
    t4x16_v[blk, :] = dots
  pltpu.sync_copy(t4x16_v, stage2.at[c, s])

  plsc.subcore_barrier()

  @pl.when(s == 0)
  def _core_reduce():
    pltpu.sync_copy(stage2.at[c], comb_v)
    inv = 1.0 / (B * K * K)
    tot = jnp.zeros((L,), jnp.float32)
    for blx in range(NQ):
      for blk in range(4):
        d = (comb_v[NQ * blx + 0, blk, :] + comb_v[NQ * blx + 1, blk, :] +
             comb_v[NQ * blx + 2, blk, :] + comb_v[NQ * blx + 3, blk, :])
        diag_here = [p - blk * L for p in DIAG_SLOTS
                     if blk * L <= p < (blk + 1) * L]
        dm = lanes < 0
        for dpos in diag_here:
          dm = dm | (lanes == dpos)
        off = jnp.where(dm, 1.0, 0.0)
        w = jnp.where(dm, 1.0, 2.0)
        tot = tot + w * jnp.abs(d - off)
    t16_v[...] = jnp.full((L,), jnp.sum(tot) * inv)
    pltpu.sync_copy(t16_v, out_hbm.at[c])


@jax.jit
def kernel(features, scores):
  table = features.reshape(B * T, D)
  mesh = plsc.VectorSubcoreMesh(core_axis_name="c", subcore_axis_name="s",
                                num_cores=2, num_subcores=16)
  outs = pl.kernel(
      _sc_diversity,
      out_type=(
          jax.ShapeDtypeStruct((2, 16, L), jnp.float32),
          jax.ShapeDtypeStruct((2, 16, L), jnp.int32),
          jax.ShapeDtypeStruct((2, 16, 4, L), jnp.float32),
          jax.ShapeDtypeStruct((2, L), jnp.float32),
      ),
      mesh=mesh,
      compiler_params=pltpu.CompilerParams(needs_layout_passes=False),
      scratch_types=[
          pltpu.VMEM((SLEN,), jnp.float32),
          pltpu.VMEM((L,), jnp.int32),
          pltpu.VMEM((L, DQ), jnp.float32),
          pltpu.VMEM((NQ, L), jnp.float32),
          pltpu.VMEM((NQ, L), jnp.int32),
          pltpu.VMEM((64, L), jnp.float32),
          pltpu.VMEM((L,), jnp.float32),
          pltpu.VMEM((4, L), jnp.float32),
          pltpu.VMEM((16, 4, L), jnp.float32),
          pltpu.SemaphoreType.DMA,
      ],
  )(table, scores)
  out = outs[3]
  return out[0, 0] + out[1, 0]


# speedup vs baseline: 10.1719x; 1.0586x over previous
import jax
import jax.numpy as jnp
from jax import lax
from jax.experimental import pallas as pl
from jax.experimental.pallas import tpu as pltpu
from jax.experimental.pallas import tpu_sc as plsc

B, T, D = 8, 4096, 2048
K = 10
L = 16
NQ = 4
SLEN = T // NQ
DQ = D // NQ
NEG = float("-inf")
INT_MAX = 2147483647

PAIRS = [(i, j) for i in range(K) for j in range(i, K)]
DIAG_SLOTS = [p for p, (i, j) in enumerate(PAIRS) if i == j]
NP_ = len(PAIRS)
GROUPS = [
    [p for p in PAIRS if p[0] < 2],
    [p for p in PAIRS if 2 <= p[0] < 5],
    [p for p in PAIRS if p[0] >= 5],
]


def _sc_diversity(feat_hbm, scores_hbm, c_band, stage2, out_hbm,
                  scores_v, idx_v, rows_v, cb_v, mat_v, t16_v, t2x16_v,
                  t4x16_v, comb_v, sem):
  c = lax.axis_index("c")
  s = lax.axis_index("s")
  lanes = lax.iota(jnp.int32, L)
  bl = s // NQ
  q = s % NQ
  b = NQ * c + bl

  pltpu.sync_copy(scores_hbm.at[b, pl.ds(q * SLEN, SLEN)], scores_v)

  def pass_body(kk, carry):
    mxv, fiv = carry
    def chunk_body(i, carry2):
      mv, iv = carry2
      v = scores_v[pl.ds(i * L, L)]
      ids = lanes + i * L
      gt = v > mv
      return jnp.where(gt, v, mv), jnp.where(gt, ids, iv)
    mv, iv = lax.fori_loop(
        0, SLEN // L, chunk_body,
        (jnp.full((L,), NEG, jnp.float32), jnp.zeros((L,), jnp.int32)),
        unroll=4)
    mx = jnp.max(mv)
    cand = jnp.where(mv == mx, iv, INT_MAX)
    idx = jnp.min(cand)
    mxv = jnp.where(lanes == kk, mx, mxv)
    fiv = jnp.where(lanes == kk, idx + q * SLEN, fiv)
    plsc.store_scatter(scores_v, [jnp.full((L,), idx, jnp.int32)],
                       jnp.full((L,), NEG, jnp.float32), mask=lanes == 0)
    return mxv, fiv

  mxv, fiv = lax.fori_loop(
      0, K, pass_body,
      (jnp.full((L,), NEG, jnp.float32), jnp.zeros((L,), jnp.int32)))
  t2x16_v[0, :] = mxv
  t2x16_v[1, :] = plsc.bitcast(fiv, jnp.float32)
  pltpu.sync_copy(t2x16_v, c_band.at[c, s])

  plsc.subcore_barrier()

  pltpu.sync_copy(c_band.at[c, pl.ds(NQ * bl, NQ)], cb_v)
  ix = [plsc.bitcast(cb_v[r, 1, :], jnp.int32) for r in range(NQ)]

  def merge_body(kk, carry):
    v0, v1, v2, v3, fiv = carry
    vs = [v0, v1, v2, v3]
    def comb(v1_, i1, v2_, i2):
      take1 = (v1_ > v2_) | ((v1_ == v2_) & (i1 < i2))
      return jnp.where(take1, v1_, v2_), jnp.where(take1, i1, i2)
    va, ia = comb(vs[0], ix[0], vs[1], ix[1])
    vb, ib = comb(vs[2], ix[2], vs[3], ix[3])
    vm, im = comb(va, ia, vb, ib)
    mx = jnp.max(vm)
    cand = jnp.where(vm == mx, im, INT_MAX)
    idx = jnp.min(cand)
    fiv = jnp.where(lanes == kk, idx, fiv)
    idx_b = jnp.full((L,), idx, jnp.int32)
    vs = [jnp.where(ix[r] == idx_b, NEG, vs[r]) for r in range(NQ)]
    return vs[0], vs[1], vs[2], vs[3], fiv

  _, _, _, _, fiv = lax.fori_loop(
      0, K, merge_body,
      (cb_v[0, 0, :], cb_v[1, 0, :], cb_v[2, 0, :], cb_v[3, 0, :],
       jnp.zeros((L,), jnp.int32)))

  idx_v[...] = jnp.where(lanes < K, fiv, 0) + b * T
  pltpu.async_copy(feat_hbm.at[idx_v, pl.ds(q * DQ, DQ)], rows_v, sem).wait()

  accs = []
  for grp in GROUPS:
    rows_needed = sorted({r for p in grp for r in p})
    def grp_body(ci, carry, grp=grp, rows_needed=rows_needed):
      base = ci * L
      v = {r: rows_v[r, pl.ds(base, L)] for r in rows_needed}
      return tuple(a + v[i] * v[j] for a, (i, j) in zip(carry, grp))
    init = tuple(jnp.zeros((L,), jnp.float32) for _ in grp)
    accs.extend(lax.fori_loop(0, DQ // L, grp_body, init, unroll=2))
  for p in range(NP_):
    mat_v[p, :] = accs[p]
  zero = jnp.zeros((L,), jnp.float32)
  for p in range(NP_, 64):
    mat_v[p, :] = zero
  for blk in range(4):
    row_ids = lanes + blk * L
    dots = jnp.zeros((L,), jnp.float32)
    for l in range(L):
      dots = dots + plsc.load_gather(
          mat_v, [row_ids, jnp.full((L,), l, jnp.int32)])
    t4x16_v[blk, :] = dots
  pltpu.sync_copy(t4x16_v, stage2.at[c, s])

  plsc.subcore_barrier()

  @pl.when(s == 0)
  def _core_reduce():
    pltpu.sync_copy(stage2.at[c], comb_v)
    inv = 1.0 / (B * K * K)
    tot = jnp.zeros((L,), jnp.float32)
    for blx in range(NQ):
      for blk in range(4):
        d = (comb_v[NQ * blx + 0, blk, :] + comb_v[NQ * blx + 1, blk, :] +
             comb_v[NQ * blx + 2, blk, :] + comb_v[NQ * blx + 3, blk, :])
        diag_here = [p - blk * L for p in DIAG_SLOTS
                     if blk * L <= p < (blk + 1) * L]
        dm = lanes < 0
        for dpos in diag_here:
          dm = dm | (lanes == dpos)
        off = jnp.where(dm, 1.0, 0.0)
        w = jnp.where(dm, 1.0, 2.0)
        tot = tot + w * jnp.abs(d - off)
    t16_v[...] = jnp.full((L,), jnp.sum(tot) * inv)
    pltpu.sync_copy(t16_v, out_hbm.at[c])


@jax.jit
def kernel(features, scores):
  table = features.reshape(B * T, D)
  mesh = plsc.VectorSubcoreMesh(core_axis_name="c", subcore_axis_name="s",
                                num_cores=2, num_subcores=16)
  outs = pl.kernel(
      _sc_diversity,
      out_type=(
          jax.ShapeDtypeStruct((2, 16, 2, L), jnp.float32),
          jax.ShapeDtypeStruct((2, 16, 4, L), jnp.float32),
          jax.ShapeDtypeStruct((2, L), jnp.float32),
      ),
      mesh=mesh,
      compiler_params=pltpu.CompilerParams(needs_layout_passes=False),
      scratch_types=[
          pltpu.VMEM((SLEN,), jnp.float32),
          pltpu.VMEM((L,), jnp.int32),
          pltpu.VMEM((L, DQ), jnp.float32),
          pltpu.VMEM((NQ, 2, L), jnp.float32),
          pltpu.VMEM((64, L), jnp.float32),
          pltpu.VMEM((L,), jnp.float32),
          pltpu.VMEM((2, L), jnp.float32),
          pltpu.VMEM((4, L), jnp.float32),
          pltpu.VMEM((16, 4, L), jnp.float32),
          pltpu.SemaphoreType.DMA,
      ],
  )(table, scores)
  out = outs[2]
  return out[0, 0] + out[1, 0]


# speedup vs baseline: 10.2439x; 1.0071x over previous
import jax
import jax.numpy as jnp
from jax import lax
from jax.experimental import pallas as pl
from jax.experimental.pallas import tpu as pltpu
from jax.experimental.pallas import tpu_sc as plsc

B, T, D = 8, 4096, 2048
K = 10
L = 16
NQ = 4
SLEN = T // NQ
DQ = D // NQ
NEG = float("-inf")
INT_MAX = 2147483647

PAIRS = [(i, j) for i in range(K) for j in range(i, K)]
DIAG_SLOTS = [p for p, (i, j) in enumerate(PAIRS) if i == j]
NP_ = len(PAIRS)
GROUPS = [
    [p for p in PAIRS if p[0] < 2],
    [p for p in PAIRS if 2 <= p[0] < 5],
    [p for p in PAIRS if p[0] >= 5],
]


def _sc_diversity(feat_hbm, scores_hbm, c_band, stage2, out_hbm,
                  scores_v, idx_v, rows_v, cb_v, mat_v, t16_v, t2x16_v,
                  t4x16_v, comb_v, sem):
  c = lax.axis_index("c")
  s = lax.axis_index("s")
  lanes = lax.iota(jnp.int32, L)
  bl = s // NQ
  q = s % NQ
  b = NQ * c + bl

  pltpu.sync_copy(scores_hbm.at[b, pl.ds(q * SLEN, SLEN)], scores_v)

  def pass_body(kk, carry):
    mxv, fiv = carry
    def chunk_body(i, carry2):
      mv, iv = carry2
      v = scores_v[pl.ds(i * L, L)]
      ids = lanes + i * L
      gt = v > mv
      return jnp.where(gt, v, mv), jnp.where(gt, ids, iv)
    mv, iv = lax.fori_loop(
        0, SLEN // L, chunk_body,
        (jnp.full((L,), NEG, jnp.float32), jnp.zeros((L,), jnp.int32)),
        unroll=8)
    mx = jnp.max(mv)
    cand = jnp.where(mv == mx, iv, INT_MAX)
    idx = jnp.min(cand)
    mxv = jnp.where(lanes == kk, mx, mxv)
    fiv = jnp.where(lanes == kk, idx + q * SLEN, fiv)
    plsc.store_scatter(scores_v, [jnp.full((L,), idx, jnp.int32)],
                       jnp.full((L,), NEG, jnp.float32), mask=lanes == 0)
    return mxv, fiv

  mxv, fiv = lax.fori_loop(
      0, K, pass_body,
      (jnp.full((L,), NEG, jnp.float32), jnp.zeros((L,), jnp.int32)))
  t2x16_v[0, :] = mxv
  t2x16_v[1, :] = plsc.bitcast(fiv, jnp.float32)
  pltpu.sync_copy(t2x16_v, c_band.at[c, s])

  plsc.subcore_barrier()

  pltpu.sync_copy(c_band.at[c, pl.ds(NQ * bl, NQ)], cb_v)
  ix = [plsc.bitcast(cb_v[r, 1, :], jnp.int32) for r in range(NQ)]

  def merge_body(kk, carry):
    v0, v1, v2, v3, fiv = carry
    vs = [v0, v1, v2, v3]
    def comb(v1_, i1, v2_, i2):
      take1 = (v1_ > v2_) | ((v1_ == v2_) & (i1 < i2))
      return jnp.where(take1, v1_, v2_), jnp.where(take1, i1, i2)
    va, ia = comb(vs[0], ix[0], vs[1], ix[1])
    vb, ib = comb(vs[2], ix[2], vs[3], ix[3])
    vm, im = comb(va, ia, vb, ib)
    mx = jnp.max(vm)
    cand = jnp.where(vm == mx, im, INT_MAX)
    idx = jnp.min(cand)
    fiv = jnp.where(lanes == kk, idx, fiv)
    idx_b = jnp.full((L,), idx, jnp.int32)
    vs = [jnp.where(ix[r] == idx_b, NEG, vs[r]) for r in range(NQ)]
    return vs[0], vs[1], vs[2], vs[3], fiv

  _, _, _, _, fiv = lax.fori_loop(
      0, K, merge_body,
      (cb_v[0, 0, :], cb_v[1, 0, :], cb_v[2, 0, :], cb_v[3, 0, :],
       jnp.zeros((L,), jnp.int32)))

  idx_v[...] = jnp.where(lanes < K, fiv, 0) + b * T
  pltpu.async_copy(feat_hbm.at[idx_v, pl.ds(q * DQ, DQ)], rows_v, sem).wait()

  accs = []
  for grp in GROUPS:
    rows_needed = sorted({r for p in grp for r in p})
    def grp_body(ci, carry, grp=grp, rows_needed=rows_needed):
      base = ci * L
      v = {r: rows_v[r, pl.ds(base, L)] for r in rows_needed}
      return tuple(a + v[i] * v[j] for a, (i, j) in zip(carry, grp))
    init = tuple(jnp.zeros((L,), jnp.float32) for _ in grp)
    accs.extend(lax.fori_loop(0, DQ // L, grp_body, init, unroll=2))
  for p in range(NP_):
    mat_v[p, :] = accs[p]
  zero = jnp.zeros((L,), jnp.float32)
  for p in range(NP_, 64):
    mat_v[p, :] = zero
  for blk in range(4):
    row_ids = lanes + blk * L
    dots = jnp.zeros((L,), jnp.float32)
    for l in range(L):
      dots = dots + plsc.load_gather(
          mat_v, [row_ids, jnp.full((L,), l, jnp.int32)])
    t4x16_v[blk, :] = dots
  pltpu.sync_copy(t4x16_v, stage2.at[c, s])

  plsc.subcore_barrier()

  @pl.when(s == 0)
  def _core_reduce():
    pltpu.sync_copy(stage2.at[c], comb_v)
    inv = 1.0 / (B * K * K)
    tot = jnp.zeros((L,), jnp.float32)
    for blx in range(NQ):
      for blk in range(4):
        d = (comb_v[NQ * blx + 0, blk, :] + comb_v[NQ * blx + 1, blk, :] +
             comb_v[NQ * blx + 2, blk, :] + comb_v[NQ * blx + 3, blk, :])
        diag_here = [p - blk * L for p in DIAG_SLOTS
                     if blk * L <= p < (blk + 1) * L]
        dm = lanes < 0
        for dpos in diag_here:
          dm = dm | (lanes == dpos)
        off = jnp.where(dm, 1.0, 0.0)
        w = jnp.where(dm, 1.0, 2.0)
        tot = tot + w * jnp.abs(d - off)
    t16_v[...] = jnp.full((L,), jnp.sum(tot) * inv)
    pltpu.sync_copy(t16_v, out_hbm.at[c])


@jax.jit
def kernel(features, scores):
  table = features.reshape(B * T, D)
  mesh = plsc.VectorSubcoreMesh(core_axis_name="c", subcore_axis_name="s",
                                num_cores=2, num_subcores=16)
  outs = pl.kernel(
      _sc_diversity,
      out_type=(
          jax.ShapeDtypeStruct((2, 16, 2, L), jnp.float32),
          jax.ShapeDtypeStruct((2, 16, 4, L), jnp.float32),
          jax.ShapeDtypeStruct((2, L), jnp.float32),
      ),
      mesh=mesh,
      compiler_params=pltpu.CompilerParams(needs_layout_passes=False),
      scratch_types=[
          pltpu.VMEM((SLEN,), jnp.float32),
          pltpu.VMEM((L,), jnp.int32),
          pltpu.VMEM((L, DQ), jnp.float32),
          pltpu.VMEM((NQ, 2, L), jnp.float32),
          pltpu.VMEM((64, L), jnp.float32),
          pltpu.VMEM((L,), jnp.float32),
          pltpu.VMEM((2, L), jnp.float32),
          pltpu.VMEM((4, L), jnp.float32),
          pltpu.VMEM((16, 4, L), jnp.float32),
          pltpu.SemaphoreType.DMA,
      ],
  )(table, scores)
  out = outs[2]
  return out[0, 0] + out[1, 0]
